# speedup vs baseline: 5.1966x; 5.1966x over previous
import functools

import jax
import jax.numpy as jnp
from jax import lax
from jax.experimental import pallas as pl
from jax.experimental.pallas import tpu as pltpu
from jax.experimental.pallas import tpu_sc as plsc

N_NODES = 10000
N_EDGES = 320000
D = 128
DEG = 32

BN = 4
BE = BN * DEG
NBLOCKS = N_NODES // BN
NW = 32
NVREG = D // 16


def _body(x_hbm, w_hbm, idx_hbm, out_hbm, idx_v, rows_v, w_v, out_v, sem):
    wid = lax.axis_index("s") * 2 + lax.axis_index("c")
    count = (NBLOCKS - wid + NW - 1) // NW

    def step(k, _):
        block = wid + NW * k
        e0 = block * BE
        pltpu.sync_copy(idx_hbm.at[pl.ds(e0, BE)], idx_v)
        pltpu.async_copy(x_hbm.at[idx_v], rows_v, sem).wait()
        pltpu.sync_copy(w_hbm.at[pl.ds(e0, BE)], w_v)

        for n in range(BN):
            def jbody(j, acc):
                e = n * DEG + j
                return tuple(
                    acc[d]
                    + rows_v[e, pl.ds(d * 16, 16)] * w_v[e, pl.ds(d * 16, 16)]
                    for d in range(NVREG)
                )

            acc = lax.fori_loop(
                0, DEG, jbody,
                tuple(jnp.zeros((16,), jnp.float32) for _ in range(NVREG)),
            )
            for d in range(NVREG):
                out_v[n, pl.ds(d * 16, 16)] = acc[d] * (1.0 / DEG)

        pltpu.sync_copy(out_v, out_hbm.at[pl.ds(block * BN, BN)])
        return 0

    lax.fori_loop(0, count, step, 0)


@jax.jit
def _graph_conv(x, w, idx):
    mesh = plsc.VectorSubcoreMesh(core_axis_name="c", subcore_axis_name="s")
    k = functools.partial(
        pl.kernel,
        mesh=mesh,
        out_type=jax.ShapeDtypeStruct((N_NODES, D), jnp.float32),
        scratch_types=[
            pltpu.VMEM((BE,), jnp.int32),
            pltpu.VMEM((BE, D), jnp.float32),
            pltpu.VMEM((BE, D), jnp.float32),
            pltpu.VMEM((BN, D), jnp.float32),
            pltpu.SemaphoreType.DMA,
        ],
    )(_body)
    return k(x, w, idx)


def kernel(input, weights, idxn):
    return _graph_conv(input, weights, idxn)


# speedup vs baseline: 7.6582x; 1.4737x over previous
import functools

import jax
import jax.numpy as jnp
from jax import lax
from jax.experimental import pallas as pl
from jax.experimental.pallas import tpu as pltpu
from jax.experimental.pallas import tpu_sc as plsc

N_NODES = 10000
N_EDGES = 320000
D = 128
DEG = 32

BN = 4
BE = BN * DEG
NBLOCKS = N_NODES // BN
NW = 32
NVREG = D // 16
KMAX = 80


def _body(x_hbm, w_hbm, idx_hbm, out_hbm,
          idx_v, rows0, rows1, w0, w1, out0, out1,
          sg0, sg1, sw0, sw1, so0, so1):
    rows = (rows0, rows1)
    wv = (w0, w1)
    outv = (out0, out1)
    sg = (sg0, sg1)
    sw = (sw0, sw1)
    so = (so0, so1)

    wid = lax.axis_index("s") * 2 + lax.axis_index("c")
    start = (wid * NBLOCKS) // NW
    end = ((wid + 1) * NBLOCKS) // NW
    count = end - start

    pltpu.sync_copy(idx_hbm.at[pl.ds(start * BE, 78 * BE)], idx_v.at[pl.ds(0, 78 * BE)])

    @pl.when(count > 78)
    def _():
        pltpu.sync_copy(idx_hbm.at[pl.ds((start + 78) * BE, BE)],
                        idx_v.at[pl.ds(78 * BE, BE)])

    def kth_block(k):
        kk = jnp.minimum(k, count - 1)
        return kk, start + kk

    def issue(k, b):
        kk, block = kth_block(k)
        pltpu.async_copy(x_hbm.at[idx_v.at[pl.ds(kk * BE, BE)]], rows[b], sg[b])
        pltpu.async_copy(w_hbm.at[pl.ds(block * BE, BE)], wv[b], sw[b])

    issue(0, 0)
    issue(1, 1)

    def step(t, _):
        for b in range(2):
            k = 2 * t + b
            kk, block = kth_block(k)
            pltpu.make_async_copy(
                x_hbm.at[idx_v.at[pl.ds(kk * BE, BE)]], rows[b], sg[b]).wait()
            pltpu.make_async_copy(
                w_hbm.at[pl.ds(block * BE, BE)], wv[b], sw[b]).wait()

            @pl.when(t >= 1)
            def _():
                pltpu.make_async_copy(
                    outv[b], out_hbm.at[pl.ds(block * BN, BN)], so[b]).wait()

            for n in range(BN):
                def jbody(j4, acc, n=n, b=b):
                    res = list(acc)
                    for u in range(8):
                        e = n * DEG + j4 * 8 + u
                        for d in range(NVREG):
                            res[d] = res[d] + (rows[b][e, pl.ds(d * 16, 16)]
                                               * wv[b][e, pl.ds(d * 16, 16)])
                    return tuple(res)

                acc = lax.fori_loop(
                    0, DEG // 8, jbody,
                    tuple(jnp.zeros((16,), jnp.float32) for _ in range(NVREG)),
                )
                for d in range(NVREG):
                    outv[b][n, pl.ds(d * 16, 16)] = acc[d] * (1.0 / DEG)

            pltpu.async_copy(outv[b], out_hbm.at[pl.ds(block * BN, BN)], so[b])

            @pl.when(t < KMAX // 2 - 1)
            def _():
                issue(k + 2, b)
        return 0

    lax.fori_loop(0, KMAX // 2, step, 0)

    for b in range(2):
        pltpu.make_async_copy(outv[b], out_hbm.at[pl.ds(0, BN)], so[b]).wait()


@jax.jit
def _graph_conv(x, w, idx):
    mesh = plsc.VectorSubcoreMesh(core_axis_name="c", subcore_axis_name="s")
    k = functools.partial(
        pl.kernel,
        mesh=mesh,
        out_type=jax.ShapeDtypeStruct((N_NODES, D), jnp.float32),
        scratch_types=[
            pltpu.VMEM((79 * BE,), jnp.int32),
            pltpu.VMEM((BE, D), jnp.float32),
            pltpu.VMEM((BE, D), jnp.float32),
            pltpu.VMEM((BE, D), jnp.float32),
            pltpu.VMEM((BE, D), jnp.float32),
            pltpu.VMEM((BN, D), jnp.float32),
            pltpu.VMEM((BN, D), jnp.float32),
            pltpu.SemaphoreType.DMA,
            pltpu.SemaphoreType.DMA,
            pltpu.SemaphoreType.DMA,
            pltpu.SemaphoreType.DMA,
            pltpu.SemaphoreType.DMA,
            pltpu.SemaphoreType.DMA,
        ],
    )(_body)
    return k(x, w, idx)


def kernel(input, weights, idxn):
    return _graph_conv(input, weights, idxn)


# speedup vs baseline: 11.9710x; 1.5632x over previous
import functools

import jax
import jax.numpy as jnp
from jax import lax
from jax.experimental import pallas as pl
from jax.experimental.pallas import tpu as pltpu
from jax.experimental.pallas import tpu_sc as plsc

N_NODES = 10000
N_EDGES = 320000
D = 128
DEG = 32

BN = 4
BE = BN * DEG
NBLOCKS = N_NODES // BN
NW = 32
NVREG = D // 16
KMAX = 80


def _body(x_hbm, w_hbm, idx_hbm, out_hbm,
          idx_v, rows0, rows1, w0, w1, out0, out1,
          sg0, sg1, sw0, sw1, so0, so1):
    rows = (rows0, rows1)
    wv = (w0, w1)
    outv = (out0, out1)
    sg = (sg0, sg1)
    sw = (sw0, sw1)
    so = (so0, so1)

    wid = lax.axis_index("s") * 2 + lax.axis_index("c")
    start = (wid * NBLOCKS) // NW
    end = ((wid + 1) * NBLOCKS) // NW
    count = end - start

    pltpu.sync_copy(idx_hbm.at[pl.ds(start * BE, 78 * BE)], idx_v.at[pl.ds(0, 78 * BE)])

    @pl.when(count > 78)
    def _():
        pltpu.sync_copy(idx_hbm.at[pl.ds((start + 78) * BE, BE)],
                        idx_v.at[pl.ds(78 * BE, BE)])

    def kth_block(k):
        kk = jnp.minimum(k, count - 1)
        return kk, start + kk

    def issue(k, b):
        kk, block = kth_block(k)
        pltpu.async_copy(x_hbm.at[idx_v.at[pl.ds(kk * BE, BE)]], rows[b], sg[b])
        pltpu.async_copy(w_hbm.at[pl.ds(block * BE, BE)], wv[b], sw[b])

    issue(0, 0)
    issue(1, 1)

    def step(t, _):
        for b in range(2):
            k = 2 * t + b
            kk, block = kth_block(k)
            pltpu.make_async_copy(
                x_hbm.at[idx_v.at[pl.ds(kk * BE, BE)]], rows[b], sg[b]).wait()
            pltpu.make_async_copy(
                w_hbm.at[pl.ds(block * BE, BE)], wv[b], sw[b]).wait()

            @pl.when(t >= 1)
            def _():
                pltpu.make_async_copy(
                    outv[b], out_hbm.at[pl.ds(block * BN, BN)], so[b]).wait()

            for n in range(BN):
                def jbody(j4, acc, n=n, b=b):
                    res = list(acc)
                    for u in range(4):
                        e = n * DEG + j4 * 4 + u
                        for d in range(NVREG):
                            res[d] = res[d] + (rows[b][e, pl.ds(d * 16, 16)]
                                               * wv[b][e, pl.ds(d * 16, 16)])
                    return tuple(res)

                acc = lax.fori_loop(
                    0, DEG // 4, jbody,
                    tuple(jnp.zeros((16,), jnp.float32) for _ in range(NVREG)),
                )
                for d in range(NVREG):
                    outv[b][n, pl.ds(d * 16, 16)] = acc[d] * (1.0 / DEG)

            pltpu.async_copy(outv[b], out_hbm.at[pl.ds(block * BN, BN)], so[b])

            @pl.when(t < KMAX // 2 - 1)
            def _():
                issue(k + 2, b)
        return 0

    lax.fori_loop(0, KMAX // 2, step, 0)

    for b in range(2):
        pltpu.make_async_copy(outv[b], out_hbm.at[pl.ds(0, BN)], so[b]).wait()


@jax.jit
def _graph_conv(x, w, idx):
    mesh = plsc.VectorSubcoreMesh(core_axis_name="c", subcore_axis_name="s")
    k = functools.partial(
        pl.kernel,
        mesh=mesh,
        out_type=jax.ShapeDtypeStruct((N_NODES, D), jnp.float32),
        scratch_types=[
            pltpu.VMEM((79 * BE,), jnp.int32),
            pltpu.VMEM((BE, D), jnp.float32),
            pltpu.VMEM((BE, D), jnp.float32),
            pltpu.VMEM((BE, D), jnp.float32),
            pltpu.VMEM((BE, D), jnp.float32),
            pltpu.VMEM((BN, D), jnp.float32),
            pltpu.VMEM((BN, D), jnp.float32),
            pltpu.SemaphoreType.DMA,
            pltpu.SemaphoreType.DMA,
            pltpu.SemaphoreType.DMA,
            pltpu.SemaphoreType.DMA,
            pltpu.SemaphoreType.DMA,
            pltpu.SemaphoreType.DMA,
        ],
    )(_body)
    return k(x, w, idx)


def kernel(input, weights, idxn):
    return _graph_conv(input, weights, idxn)


# speedup vs baseline: 12.9246x; 1.0797x over previous
import functools

import jax
import jax.numpy as jnp
from jax import lax
from jax.experimental import pallas as pl
from jax.experimental.pallas import tpu as pltpu
from jax.experimental.pallas import tpu_sc as plsc

N_NODES = 10000
N_EDGES = 320000
D = 128
DEG = 32

BN = 4
BE = BN * DEG
NBLOCKS = N_NODES // BN
NW = 32
NVREG = D // 16
NBUF = 3
KMAX = 81


def _body(x_hbm, w_hbm, idx_hbm, out_hbm,
          idx_v, rows0, rows1, rows2, w0, w1, w2, out0, out1, out2,
          sg0, sg1, sg2, sw0, sw1, sw2, so0, so1, so2):
    rows = (rows0, rows1, rows2)
    wv = (w0, w1, w2)
    outv = (out0, out1, out2)
    sg = (sg0, sg1, sg2)
    sw = (sw0, sw1, sw2)
    so = (so0, so1, so2)

    wid = lax.axis_index("s") * 2 + lax.axis_index("c")
    start = (wid * NBLOCKS) // NW
    end = ((wid + 1) * NBLOCKS) // NW
    count = end - start

    pltpu.sync_copy(idx_hbm.at[pl.ds(start * BE, 78 * BE)], idx_v.at[pl.ds(0, 78 * BE)])

    @pl.when(count > 78)
    def _():
        pltpu.sync_copy(idx_hbm.at[pl.ds((start + 78) * BE, BE)],
                        idx_v.at[pl.ds(78 * BE, BE)])

    def kth_block(k):
        kk = jnp.minimum(k, count - 1)
        return kk, start + kk

    def issue(k, b):
        kk, block = kth_block(k)
        pltpu.async_copy(x_hbm.at[idx_v.at[pl.ds(kk * BE, BE)]], rows[b], sg[b])
        pltpu.async_copy(w_hbm.at[pl.ds(block * BE, BE)], wv[b], sw[b])

    for b in range(NBUF):
        issue(b, b)

    def step(t, _):
        for b in range(NBUF):
            k = NBUF * t + b
            kk, block = kth_block(k)
            pltpu.make_async_copy(
                x_hbm.at[idx_v.at[pl.ds(kk * BE, BE)]], rows[b], sg[b]).wait()
            pltpu.make_async_copy(
                w_hbm.at[pl.ds(block * BE, BE)], wv[b], sw[b]).wait()

            @pl.when(t >= 1)
            def _():
                pltpu.make_async_copy(
                    outv[b], out_hbm.at[pl.ds(block * BN, BN)], so[b]).wait()

            for n in range(BN):
                def jbody(j4, acc, n=n, b=b):
                    res = list(acc)
                    for u in range(4):
                        e = n * DEG + j4 * 4 + u
                        for d in range(NVREG):
                            res[d] = res[d] + (rows[b][e, pl.ds(d * 16, 16)]
                                               * wv[b][e, pl.ds(d * 16, 16)])
                    return tuple(res)

                acc = lax.fori_loop(
                    0, DEG // 4, jbody,
                    tuple(jnp.zeros((16,), jnp.float32) for _ in range(NVREG)),
                )
                for d in range(NVREG):
                    outv[b][n, pl.ds(d * 16, 16)] = acc[d] * (1.0 / DEG)

            pltpu.async_copy(outv[b], out_hbm.at[pl.ds(block * BN, BN)], so[b])

            @pl.when(t < KMAX // NBUF - 1)
            def _():
                issue(k + NBUF, b)
        return 0

    lax.fori_loop(0, KMAX // NBUF, step, 0)

    for b in range(NBUF):
        pltpu.make_async_copy(outv[b], out_hbm.at[pl.ds(0, BN)], so[b]).wait()


@jax.jit
def _graph_conv(x, w, idx):
    mesh = plsc.VectorSubcoreMesh(core_axis_name="c", subcore_axis_name="s")
    k = functools.partial(
        pl.kernel,
        mesh=mesh,
        out_type=jax.ShapeDtypeStruct((N_NODES, D), jnp.float32),
        scratch_types=[
            pltpu.VMEM((79 * BE,), jnp.int32),
            pltpu.VMEM((BE, D), jnp.float32),
            pltpu.VMEM((BE, D), jnp.float32),
            pltpu.VMEM((BE, D), jnp.float32),
            pltpu.VMEM((BE, D), jnp.float32),
            pltpu.VMEM((BE, D), jnp.float32),
            pltpu.VMEM((BE, D), jnp.float32),
            pltpu.VMEM((BN, D), jnp.float32),
            pltpu.VMEM((BN, D), jnp.float32),
            pltpu.VMEM((BN, D), jnp.float32),
            pltpu.SemaphoreType.DMA,
            pltpu.SemaphoreType.DMA,
            pltpu.SemaphoreType.DMA,
            pltpu.SemaphoreType.DMA,
            pltpu.SemaphoreType.DMA,
            pltpu.SemaphoreType.DMA,
            pltpu.SemaphoreType.DMA,
            pltpu.SemaphoreType.DMA,
            pltpu.SemaphoreType.DMA,
        ],
    )(_body)
    return k(x, w, idx)


def kernel(input, weights, idxn):
    return _graph_conv(input, weights, idxn)


# speedup vs baseline: 15.9646x; 1.2352x over previous
import functools

import jax
import jax.numpy as jnp
from jax import lax
from jax.experimental import pallas as pl
from jax.experimental.pallas import tpu as pltpu
from jax.experimental.pallas import tpu_sc as plsc

N_NODES = 10000
N_EDGES = 320000
D = 128
DEG = 32

BN = 4
BE = BN * DEG
NBLOCKS = N_NODES // BN
NW = 32
NCHUNK = D // 32
NBUF = 3
KMAX = 81
_HI = -65536


def _body(x_hbm, w_hbm, idx_hbm, out_hbm,
          idx_v, rows0, rows1, rows2, w0, w1, w2, out0, out1, out2,
          sg0, sg1, sg2, sw0, sw1, sw2, so0, so1, so2):
    rows = (rows0, rows1, rows2)
    wv = (w0, w1, w2)
    outv = (out0, out1, out2)
    sg = (sg0, sg1, sg2)
    sw = (sw0, sw1, sw2)
    so = (so0, so1, so2)

    wid = lax.axis_index("s") * 2 + lax.axis_index("c")
    start = (wid * NBLOCKS) // NW
    end = ((wid + 1) * NBLOCKS) // NW
    count = end - start

    pltpu.sync_copy(idx_hbm.at[pl.ds(start * BE, 78 * BE)], idx_v.at[pl.ds(0, 78 * BE)])

    @pl.when(count > 78)
    def _():
        pltpu.sync_copy(idx_hbm.at[pl.ds((start + 78) * BE, BE)],
                        idx_v.at[pl.ds(78 * BE, BE)])

    def kth_block(k):
        kk = jnp.minimum(k, count - 1)
        return kk, start + kk

    def issue(k, b):
        kk, block = kth_block(k)
        pltpu.async_copy(x_hbm.at[idx_v.at[pl.ds(kk * BE, BE)]], rows[b], sg[b])
        pltpu.async_copy(w_hbm.at[pl.ds(block * BE, BE)], wv[b], sw[b])

    for b in range(NBUF):
        issue(b, b)

    def step(t, _):
        for b in range(NBUF):
            k = NBUF * t + b
            kk, block = kth_block(k)
            pltpu.make_async_copy(
                x_hbm.at[idx_v.at[pl.ds(kk * BE, BE)]], rows[b], sg[b]).wait()
            pltpu.make_async_copy(
                w_hbm.at[pl.ds(block * BE, BE)], wv[b], sw[b]).wait()

            @pl.when(t >= 1)
            def _():
                pltpu.make_async_copy(
                    outv[b], out_hbm.at[pl.ds(block * BN, BN)], so[b]).wait()

            for n in range(BN):
                def jbody(j4, acc, n=n, b=b):
                    res = list(acc)
                    for u in range(4):
                        e = n * DEG + j4 * 4 + u
                        for c in range(NCHUNK):
                            ri = rows[b][e, pl.ds(c * 16, 16)]
                            ra = lax.bitcast_convert_type(ri << 16, jnp.float32)
                            rb = lax.bitcast_convert_type(ri & _HI, jnp.float32)
                            res[2 * c] = res[2 * c] + ra * wv[b][
                                e, pl.ds(c * 32, 16)]
                            res[2 * c + 1] = res[2 * c + 1] + rb * wv[b][
                                e, pl.ds(c * 32 + 16, 16)]
                    return tuple(res)

                acc = lax.fori_loop(
                    0, DEG // 4, jbody,
                    tuple(jnp.zeros((16,), jnp.float32) for _ in range(2 * NCHUNK)),
                )
                for h in range(2 * NCHUNK):
                    outv[b][n, pl.ds(h * 16, 16)] = acc[h] * (1.0 / DEG)

            pltpu.async_copy(outv[b], out_hbm.at[pl.ds(block * BN, BN)], so[b])

            @pl.when(t < KMAX // NBUF - 1)
            def _():
                issue(k + NBUF, b)
        return 0

    lax.fori_loop(0, KMAX // NBUF, step, 0)

    for b in range(NBUF):
        pltpu.make_async_copy(outv[b], out_hbm.at[pl.ds(0, BN)], so[b]).wait()


@jax.jit
def _graph_conv(x, w, idx):
    xp = x.reshape(N_NODES, NCHUNK, 2, 16).transpose(0, 1, 3, 2)
    xp = xp.reshape(N_NODES, D // 2, 2).astype(jnp.bfloat16)
    xp = jax.lax.bitcast_convert_type(xp, jnp.int32)
    mesh = plsc.VectorSubcoreMesh(core_axis_name="c", subcore_axis_name="s")
    k = functools.partial(
        pl.kernel,
        mesh=mesh,
        compiler_params=pltpu.CompilerParams(use_tc_tiling_on_sc=False),
        out_type=jax.ShapeDtypeStruct((N_NODES, D), jnp.float32),
        scratch_types=[
            pltpu.VMEM((79 * BE,), jnp.int32),
            pltpu.VMEM((BE, D // 2), jnp.int32),
            pltpu.VMEM((BE, D // 2), jnp.int32),
            pltpu.VMEM((BE, D // 2), jnp.int32),
            pltpu.VMEM((BE, D), jnp.float32),
            pltpu.VMEM((BE, D), jnp.float32),
            pltpu.VMEM((BE, D), jnp.float32),
            pltpu.VMEM((BN, D), jnp.float32),
            pltpu.VMEM((BN, D), jnp.float32),
            pltpu.VMEM((BN, D), jnp.float32),
            pltpu.SemaphoreType.DMA,
            pltpu.SemaphoreType.DMA,
            pltpu.SemaphoreType.DMA,
            pltpu.SemaphoreType.DMA,
            pltpu.SemaphoreType.DMA,
            pltpu.SemaphoreType.DMA,
            pltpu.SemaphoreType.DMA,
            pltpu.SemaphoreType.DMA,
            pltpu.SemaphoreType.DMA,
        ],
    )(_body)
    return k(xp, w, idx)


def kernel(input, weights, idxn):
    return _graph_conv(input, weights, idxn)
